# baseline (speedup 1.0000x reference)
import functools

import jax
import jax.numpy as jnp
from jax import lax
from jax.experimental import pallas as pl
from jax.experimental.pallas import tpu as pltpu
from jax.experimental.pallas import tpu_sc as plsc

NC = 2
NS = 16
K = 128
BR = 256


def _round_up(a, b):
    return (a + b - 1) // b * b


def _sc_mesh():
    return plsc.VectorSubcoreMesh(core_axis_name="c", subcore_axis_name="s")


@functools.cache
def _make_deg_kernel(N_pad, E_pad):
    C = E_pad // (NC * NS * K)
    R = N_pad // NS

    def body(dst_hbm, consts_hbm, out_hbm, didx, cbuf, acc):
        cid = lax.axis_index("c")
        sid = lax.axis_index("s")
        pltpu.sync_copy(consts_hbm.at[0], cbuf)

        @pl.loop(0, R // K)
        def _(j):
            pltpu.sync_copy(cbuf, acc.at[pl.ds(sid * R + j * K, K)])

        pltpu.sync_copy(consts_hbm.at[1], cbuf)
        plsc.subcore_barrier()

        base = (cid * NS + sid) * C * K

        @pl.loop(0, C)
        def _(j):
            pltpu.sync_copy(dst_hbm.at[pl.ds(base + j * K, K)], didx)
            pltpu.sync_copy(cbuf, acc.at[didx], add=True)

        plsc.subcore_barrier()

        @pl.loop(0, R // K)
        def _(j):
            r0 = sid * R + j * K
            pltpu.sync_copy(acc.at[pl.ds(r0, K)], cbuf)
            pltpu.sync_copy(cbuf, out_hbm.at[pl.ds(cid * N_pad + r0, K)])

    return pl.kernel(
        body,
        out_type=jax.ShapeDtypeStruct((NC * N_pad, 128), jnp.float32),
        mesh=_sc_mesh(),
        scratch_types=[
            pltpu.VMEM((K,), jnp.int32),
            pltpu.VMEM((K, 128), jnp.float32),
            pltpu.VMEM_SHARED((N_pad, 128), jnp.float32),
        ],
    )


NBUF = 2
NIS = 4


@functools.cache
def _make_edge_kernel(N_pad, C0, C1, H):
    R = N_pad // NS

    def body(ys_hbm, src_hbm, dst_hbm, zeros_hbm, out_hbm,
             sidx, didx, rows0, rows1, acc, gsem, ssem, isem):
        cid = lax.axis_index("c")
        sid = lax.axis_index("s")
        rowsb = (rows0, rows1)

        def make_pipeline(Cc, base):
            def start_idx(j):
                sl = j % NIS
                pltpu.async_copy(src_hbm.at[pl.ds(base + j * K, K)],
                                 sidx.at[sl], isem.at[sl])
                pltpu.async_copy(dst_hbm.at[pl.ds(base + j * K, K)],
                                 didx.at[sl], isem.at[sl])

            def wait_idx(j):
                sl = j % NIS
                pltpu.make_async_copy(src_hbm.at[pl.ds(0, K)], sidx.at[sl],
                                      isem.at[sl]).wait()
                pltpu.make_async_copy(dst_hbm.at[pl.ds(0, K)], didx.at[sl],
                                      isem.at[sl]).wait()

            def start_gather(b, j):
                pltpu.async_copy(ys_hbm.at[sidx.at[j % NIS]], rowsb[b],
                                 gsem.at[b])

            def wait_gather(b):
                pltpu.make_async_copy(ys_hbm.at[sidx.at[0]], rowsb[b],
                                      gsem.at[b]).wait()

            def start_scatter(b, j):
                pltpu.async_copy(rowsb[b], acc.at[didx.at[j % NIS]],
                                 ssem.at[b], add=True)

            def wait_scatter(b):
                pltpu.make_async_copy(rowsb[b], acc.at[didx.at[0]],
                                      ssem.at[b]).wait()

            for j in range(NIS):
                start_idx(j)
            for b in range(NBUF):
                wait_idx(b)
                start_gather(b, b)

            @pl.loop(0, Cc // NBUF)
            def _(q):
                for b in range(NBUF):
                    wait_gather(b)
                    start_scatter(b, q * NBUF + b)
                for b in range(NBUF):
                    jn = q * NBUF + NBUF + b
                    jf = q * NBUF + 2 * NBUF + b

                    @pl.when(jn < Cc)
                    def _():
                        wait_scatter(b)

                        @pl.when(jf < Cc)
                        def _():
                            start_idx(jf)

                        wait_idx(jn)
                        start_gather(b, jn)

            for b in range(NBUF):
                wait_scatter(b)

        pltpu.sync_copy(zeros_hbm, rows0)

        @pl.loop(0, R // K)
        def _(j):
            pltpu.sync_copy(rows0, acc.at[pl.ds(sid * R + j * K, K)])

        plsc.subcore_barrier()

        @pl.when(cid == 0)
        def _():
            make_pipeline(C0, sid * C0 * K)

        if C1 > 0:
            @pl.when(cid == 1)
            def _():
                make_pipeline(C1, NS * C0 * K + sid * C1 * K)

        plsc.subcore_barrier()

        @pl.loop(0, R // K)
        def _(j):
            r0 = sid * R + j * K
            pltpu.sync_copy(acc.at[pl.ds(r0, K)], rows0)
            pltpu.sync_copy(rows0, out_hbm.at[pl.ds(cid * N_pad + r0, K)])

    return pl.kernel(
        body,
        out_type=jax.ShapeDtypeStruct((NC * N_pad, H), jnp.float32),
        mesh=_sc_mesh(),
        scratch_types=[
            pltpu.VMEM((NIS, K), jnp.int32),
            pltpu.VMEM((NIS, K), jnp.int32),
            pltpu.VMEM((K, H), jnp.float32),
            pltpu.VMEM((K, H), jnp.float32),
            pltpu.VMEM_SHARED((N_pad, H), jnp.float32),
            pltpu.SemaphoreType.DMA((NBUF,)),
            pltpu.SemaphoreType.DMA((NBUF,)),
            pltpu.SemaphoreType.DMA((NIS,)),
        ],
    )


def _dinv_block(degp_ref, i, N):
    deg = degp_ref[0, :, 0:1] + degp_ref[1, :, 0:1] + 1.0
    row = i * BR + lax.broadcasted_iota(jnp.int32, (BR, 1), 0)
    return lax.rsqrt(deg) * (row < N).astype(jnp.float32)


@functools.cache
def _make_tc_first(N, N_pad, D, H):
    def body(x_ref, w_ref, degp_ref, o_ref):
        dinv = _dinv_block(degp_ref, pl.program_id(0), N)
        o_ref[...] = dinv * jnp.dot(x_ref[...], w_ref[...],
                                    preferred_element_type=jnp.float32)

    return pl.pallas_call(
        body,
        grid=(N_pad // BR,),
        in_specs=[
            pl.BlockSpec((BR, D), lambda i: (i, 0)),
            pl.BlockSpec((D, H), lambda i: (0, 0)),
            pl.BlockSpec((2, BR, 128), lambda i: (0, i, 0)),
        ],
        out_specs=pl.BlockSpec((BR, H), lambda i: (i, 0)),
        out_shape=jax.ShapeDtypeStruct((N_pad, H), jnp.float32),
    )


@functools.cache
def _make_tc_mid(N, N_pad, H):
    def body(p_ref, ys_ref, w_ref, b_ref, degp_ref, o_ref):
        dinv = _dinv_block(degp_ref, pl.program_id(0), N)
        agg = p_ref[0] + p_ref[1] + ys_ref[...]
        h = jnp.maximum(dinv * agg + b_ref[...], 0.0)
        o_ref[...] = dinv * jnp.dot(h, w_ref[...],
                                    preferred_element_type=jnp.float32)

    return pl.pallas_call(
        body,
        grid=(N_pad // BR,),
        in_specs=[
            pl.BlockSpec((2, BR, H), lambda i: (0, i, 0)),
            pl.BlockSpec((BR, H), lambda i: (i, 0)),
            pl.BlockSpec((H, H), lambda i: (0, 0)),
            pl.BlockSpec((1, H), lambda i: (0, 0)),
            pl.BlockSpec((2, BR, 128), lambda i: (0, i, 0)),
        ],
        out_specs=pl.BlockSpec((BR, H), lambda i: (i, 0)),
        out_shape=jax.ShapeDtypeStruct((N_pad, H), jnp.float32),
    )


@functools.cache
def _make_tc_final(N, N_pad, H):
    def body(p_ref, ys_ref, b_ref, degp_ref, o_ref):
        dinv = _dinv_block(degp_ref, pl.program_id(0), N)
        agg = p_ref[0] + p_ref[1] + ys_ref[...]
        o_ref[...] = dinv * agg + b_ref[...]

    return pl.pallas_call(
        body,
        grid=(N_pad // BR,),
        in_specs=[
            pl.BlockSpec((2, BR, H), lambda i: (0, i, 0)),
            pl.BlockSpec((BR, H), lambda i: (i, 0)),
            pl.BlockSpec((1, H), lambda i: (0, 0)),
            pl.BlockSpec((2, BR, 128), lambda i: (0, i, 0)),
        ],
        out_specs=pl.BlockSpec((BR, H), lambda i: (i, 0)),
        out_shape=jax.ShapeDtypeStruct((N_pad, H), jnp.float32),
    )


def kernel(x, edge_index, arg0, arg1, W0, b0, W1, b1, W2, b2):
    N, D = x.shape
    H = W0.shape[1]
    E = edge_index.shape[1]
    NW = NC * NS
    N_pad = _round_up(N + 1, NS * K)
    CT = _round_up(_round_up(E, NW * K) // (NW * K) * NC, 2 * NBUF)
    C0 = CT
    C1 = 0
    E_pad = CT * NS * K

    src = edge_index[0].astype(jnp.int32)
    dst = edge_index[1].astype(jnp.int32)
    if E_pad > E:
        padv = jnp.full((E_pad - E,), N, jnp.int32)
        src = jnp.concatenate([src, padv])
        dst = jnp.concatenate([dst, padv])

    x_p = jnp.pad(x.astype(jnp.float32), ((0, N_pad - N), (0, 0)))
    consts = jnp.stack([jnp.zeros((K, 128), jnp.float32),
                        jnp.ones((K, 128), jnp.float32)])
    zeros_kh = jnp.zeros((K, H), jnp.float32)

    degp = _make_deg_kernel(N_pad, E_pad)(dst, consts).reshape(NC, N_pad, 128)

    edge = _make_edge_kernel(N_pad, C0, C1, H)
    mid = _make_tc_mid(N, N_pad, H)

    ys = _make_tc_first(N, N_pad, D, H)(x_p, W0.astype(jnp.float32), degp)
    p = edge(ys, src, dst, zeros_kh).reshape(NC, N_pad, H)
    ys = mid(p, ys, W1.astype(jnp.float32), b0.reshape(1, H), degp)
    p = edge(ys, src, dst, zeros_kh).reshape(NC, N_pad, H)
    ys = mid(p, ys, W2.astype(jnp.float32), b1.reshape(1, H), degp)
    p = edge(ys, src, dst, zeros_kh).reshape(NC, N_pad, H)
    out = _make_tc_final(N, N_pad, H)(p, ys, b2.reshape(1, H), degp)

    return out[:N], arg1


# speedup vs baseline: 3.0974x; 3.0974x over previous
import functools

import jax
import jax.numpy as jnp
from jax import lax
from jax.experimental import pallas as pl
from jax.experimental.pallas import tpu as pltpu
from jax.experimental.pallas import tpu_sc as plsc

NC = 2
NS = 16
K = 128
BR = 256


def _round_up(a, b):
    return (a + b - 1) // b * b


def _sc_mesh():
    return plsc.VectorSubcoreMesh(core_axis_name="c", subcore_axis_name="s")


@functools.cache
def _make_deg_kernel(N_pad, E_pad):
    C = E_pad // (NC * NS * K)
    R = N_pad // NS

    def body(dst_hbm, consts_hbm, out_hbm, didx, cbuf, acc):
        cid = lax.axis_index("c")
        sid = lax.axis_index("s")
        pltpu.sync_copy(consts_hbm.at[0], cbuf)

        @pl.loop(0, R // K)
        def _(j):
            pltpu.sync_copy(cbuf, acc.at[pl.ds(sid * R + j * K, K)])

        pltpu.sync_copy(consts_hbm.at[1], cbuf)
        plsc.subcore_barrier()

        base = (cid * NS + sid) * C * K

        @pl.loop(0, C)
        def _(j):
            pltpu.sync_copy(dst_hbm.at[pl.ds(base + j * K, K)], didx)
            pltpu.sync_copy(cbuf, acc.at[didx], add=True)

        plsc.subcore_barrier()

        @pl.loop(0, R // K)
        def _(j):
            r0 = sid * R + j * K
            pltpu.sync_copy(acc.at[pl.ds(r0, K)], cbuf)
            pltpu.sync_copy(cbuf, out_hbm.at[pl.ds(cid * N_pad + r0, K)])

    return pl.kernel(
        body,
        out_type=jax.ShapeDtypeStruct((NC * N_pad, 128), jnp.float32),
        mesh=_sc_mesh(),
        scratch_types=[
            pltpu.VMEM((K,), jnp.int32),
            pltpu.VMEM((K, 128), jnp.float32),
            pltpu.VMEM_SHARED((N_pad, 128), jnp.float32),
        ],
    )


NBUF = 2
NIS = 4


@functools.cache
def _make_edge_kernel(N_pad, C0, C1, H):
    R = N_pad // NS

    def body(ys_hbm, src_hbm, dst_hbm, zeros_hbm, out_hbm,
             sidx, didx, rows0, rows1, acc, gsem, ssem, isem):
        cid = lax.axis_index("c")
        sid = lax.axis_index("s")
        rowsb = (rows0, rows1)

        def make_pipeline(Cc, base):
            def start_idx(j):
                sl = j % NIS
                pltpu.async_copy(src_hbm.at[pl.ds(base + j * K, K)],
                                 sidx.at[sl], isem.at[sl])
                pltpu.async_copy(dst_hbm.at[pl.ds(base + j * K, K)],
                                 didx.at[sl], isem.at[sl])

            def wait_idx(j):
                sl = j % NIS
                pltpu.make_async_copy(src_hbm.at[pl.ds(0, K)], sidx.at[sl],
                                      isem.at[sl]).wait()
                pltpu.make_async_copy(dst_hbm.at[pl.ds(0, K)], didx.at[sl],
                                      isem.at[sl]).wait()

            def start_gather(b, j):
                pltpu.async_copy(ys_hbm.at[sidx.at[j % NIS]], rowsb[b],
                                 gsem.at[b])

            def wait_gather(b):
                pltpu.make_async_copy(ys_hbm.at[sidx.at[0]], rowsb[b],
                                      gsem.at[b]).wait()

            def start_scatter(b, j):
                pltpu.async_copy(rowsb[b], acc.at[didx.at[j % NIS]],
                                 ssem.at[b], add=True)

            def wait_scatter(b):
                pltpu.make_async_copy(rowsb[b], acc.at[didx.at[0]],
                                      ssem.at[b]).wait()

            for j in range(NIS):
                start_idx(j)
            for b in range(NBUF):
                wait_idx(b)
                start_gather(b, b)

            @pl.loop(0, Cc // NBUF)
            def _(q):
                for b in range(NBUF):
                    wait_gather(b)
                    start_scatter(b, q * NBUF + b)
                for b in range(NBUF):
                    jn = q * NBUF + NBUF + b
                    jf = q * NBUF + 2 * NBUF + b

                    @pl.when(jn < Cc)
                    def _():
                        wait_scatter(b)

                        @pl.when(jf < Cc)
                        def _():
                            start_idx(jf)

                        wait_idx(jn)
                        start_gather(b, jn)

            for b in range(NBUF):
                wait_scatter(b)

        pltpu.sync_copy(zeros_hbm, rows0)

        @pl.loop(0, R // K)
        def _(j):
            pltpu.sync_copy(rows0, acc.at[pl.ds(sid * R + j * K, K)])

        plsc.subcore_barrier()

        @pl.when(cid == 0)
        def _():
            make_pipeline(C0, sid * C0 * K)

        if C1 > 0:
            @pl.when(cid == 1)
            def _():
                make_pipeline(C1, NS * C0 * K + sid * C1 * K)

        plsc.subcore_barrier()

        @pl.loop(0, R // K)
        def _(j):
            r0 = sid * R + j * K
            pltpu.sync_copy(acc.at[pl.ds(r0, K)], rows0)
            pltpu.sync_copy(rows0, out_hbm.at[pl.ds(cid * N_pad + r0, K)])

    return pl.kernel(
        body,
        out_type=jax.ShapeDtypeStruct((NC * N_pad, H), jnp.float32),
        mesh=_sc_mesh(),
        scratch_types=[
            pltpu.VMEM((NIS, K), jnp.int32),
            pltpu.VMEM((NIS, K), jnp.int32),
            pltpu.VMEM((K, H), jnp.float32),
            pltpu.VMEM((K, H), jnp.float32),
            pltpu.VMEM_SHARED((N_pad, H), jnp.float32),
            pltpu.SemaphoreType.DMA((NBUF,)),
            pltpu.SemaphoreType.DMA((NBUF,)),
            pltpu.SemaphoreType.DMA((NIS,)),
        ],
    )


def _dinv_block(degp_ref, i, N):
    deg = degp_ref[0, :, 0:1] + degp_ref[1, :, 0:1] + 1.0
    row = i * BR + lax.broadcasted_iota(jnp.int32, (BR, 1), 0)
    return lax.rsqrt(deg) * (row < N).astype(jnp.float32)


@functools.cache
def _make_tc_first(N, N_pad, D, H):
    def body(x_ref, w_ref, degp_ref, o_ref):
        dinv = _dinv_block(degp_ref, pl.program_id(0), N)
        o_ref[...] = dinv * jnp.dot(x_ref[...], w_ref[...],
                                    preferred_element_type=jnp.float32)

    return pl.pallas_call(
        body,
        grid=(N_pad // BR,),
        in_specs=[
            pl.BlockSpec((BR, D), lambda i: (i, 0)),
            pl.BlockSpec((D, H), lambda i: (0, 0)),
            pl.BlockSpec((2, BR, 128), lambda i: (0, i, 0)),
        ],
        out_specs=pl.BlockSpec((BR, H), lambda i: (i, 0)),
        out_shape=jax.ShapeDtypeStruct((N_pad, H), jnp.float32),
    )


@functools.cache
def _make_tc_mid(N, N_pad, H):
    def body(p_ref, ys_ref, w_ref, b_ref, degp_ref, o_ref):
        dinv = _dinv_block(degp_ref, pl.program_id(0), N)
        agg = p_ref[0] + p_ref[1] + ys_ref[...]
        h = jnp.maximum(dinv * agg + b_ref[...], 0.0)
        o_ref[...] = dinv * jnp.dot(h, w_ref[...],
                                    preferred_element_type=jnp.float32)

    return pl.pallas_call(
        body,
        grid=(N_pad // BR,),
        in_specs=[
            pl.BlockSpec((2, BR, H), lambda i: (0, i, 0)),
            pl.BlockSpec((BR, H), lambda i: (i, 0)),
            pl.BlockSpec((H, H), lambda i: (0, 0)),
            pl.BlockSpec((1, H), lambda i: (0, 0)),
            pl.BlockSpec((2, BR, 128), lambda i: (0, i, 0)),
        ],
        out_specs=pl.BlockSpec((BR, H), lambda i: (i, 0)),
        out_shape=jax.ShapeDtypeStruct((N_pad, H), jnp.float32),
    )


@functools.cache
def _make_tc_final(N, N_pad, H):
    def body(p_ref, ys_ref, b_ref, degp_ref, o_ref):
        dinv = _dinv_block(degp_ref, pl.program_id(0), N)
        agg = p_ref[0] + p_ref[1] + ys_ref[...]
        o_ref[...] = dinv * agg + b_ref[...]

    return pl.pallas_call(
        body,
        grid=(N_pad // BR,),
        in_specs=[
            pl.BlockSpec((2, BR, H), lambda i: (0, i, 0)),
            pl.BlockSpec((BR, H), lambda i: (i, 0)),
            pl.BlockSpec((1, H), lambda i: (0, 0)),
            pl.BlockSpec((2, BR, 128), lambda i: (0, i, 0)),
        ],
        out_specs=pl.BlockSpec((BR, H), lambda i: (i, 0)),
        out_shape=jax.ShapeDtypeStruct((N_pad, H), jnp.float32),
    )


def kernel(x, edge_index, arg0, arg1, W0, b0, W1, b1, W2, b2):
    N, D = x.shape
    H = W0.shape[1]
    E = edge_index.shape[1]
    NW = NC * NS
    N_pad = _round_up(N + 1, NS * K)
    CT = _round_up(_round_up(E, NW * K) // (NW * K) * NC, 2 * NBUF)
    C0 = CT // 2
    C1 = CT - C0
    E_pad = CT * NS * K

    src = edge_index[0].astype(jnp.int32)
    dst = edge_index[1].astype(jnp.int32)
    if E_pad > E:
        padv = N + jnp.arange(E_pad - E, dtype=jnp.int32) % K
        src = jnp.concatenate([src, padv])
        dst = jnp.concatenate([dst, padv])

    x_p = jnp.pad(x.astype(jnp.float32), ((0, N_pad - N), (0, 0)))
    consts = jnp.stack([jnp.zeros((K, 128), jnp.float32),
                        jnp.ones((K, 128), jnp.float32)])
    zeros_kh = jnp.zeros((K, H), jnp.float32)

    degp = _make_deg_kernel(N_pad, E_pad)(dst, consts).reshape(NC, N_pad, 128)

    edge = _make_edge_kernel(N_pad, C0, C1, H)
    mid = _make_tc_mid(N, N_pad, H)

    ys = _make_tc_first(N, N_pad, D, H)(x_p, W0.astype(jnp.float32), degp)
    p = edge(ys, src, dst, zeros_kh).reshape(NC, N_pad, H)
    ys = mid(p, ys, W1.astype(jnp.float32), b0.reshape(1, H), degp)
    p = edge(ys, src, dst, zeros_kh).reshape(NC, N_pad, H)
    ys = mid(p, ys, W2.astype(jnp.float32), b1.reshape(1, H), degp)
    p = edge(ys, src, dst, zeros_kh).reshape(NC, N_pad, H)
    out = _make_tc_final(N, N_pad, H)(p, ys, b2.reshape(1, H), degp)

    return out[:N], arg1


# speedup vs baseline: 3.5505x; 1.1463x over previous
import functools

import jax
import jax.numpy as jnp
from jax import lax
from jax.experimental import pallas as pl
from jax.experimental.pallas import tpu as pltpu
from jax.experimental.pallas import tpu_sc as plsc

NC = 2
NS = 16
K = 128
BR = 256


def _round_up(a, b):
    return (a + b - 1) // b * b


def _sc_mesh():
    return plsc.VectorSubcoreMesh(core_axis_name="c", subcore_axis_name="s")


@functools.cache
def _make_deg_kernel(N_pad, E_pad):
    C = E_pad // (NC * NS * K)
    NR = N_pad // 128

    def body(dst_hbm, zeros_hbm, out_hbm, didx_all, hist, rowids, acc):
        cid = lax.axis_index("c")
        sid = lax.axis_index("s")

        pltpu.sync_copy(zeros_hbm.at[pl.ds(0, NR)], hist)

        @pl.when(sid == 0)
        def _():
            pltpu.sync_copy(zeros_hbm.at[pl.ds(0, NR)], acc)

        @pl.loop(0, NR // 16)
        def _(g):
            rowids[pl.ds(g * 16, 16)] = (
                lax.iota(jnp.int32, 16) + g * 16)

        base = (cid * NS + sid) * C * K
        pltpu.sync_copy(dst_hbm.at[pl.ds(base, C * K)], didx_all)

        ones = jnp.ones((16,), jnp.float32)

        @pl.loop(0, C * K // 16)
        def _(g):
            iv = didx_all[pl.ds(g * 16, 16)]
            plsc.addupdate_scatter(hist, [iv // 128, iv % 128], ones)

        plsc.subcore_barrier()
        pltpu.sync_copy(hist, acc.at[rowids], add=True)
        plsc.subcore_barrier()

        @pl.when(sid == 0)
        def _():
            pltpu.sync_copy(acc, out_hbm.at[pl.ds(cid * NR, NR)])

    cp = pltpu.CompilerParams()
    if "needs_layout_passes" in pltpu.CompilerParams.__dataclass_fields__:
        import dataclasses as _dc
        cp = _dc.replace(cp, needs_layout_passes=False)

    return pl.kernel(
        body,
        out_type=jax.ShapeDtypeStruct((NC * NR, 128), jnp.float32),
        mesh=_sc_mesh(),
        compiler_params=cp,
        scratch_types=[
            pltpu.VMEM((C * K,), jnp.int32),
            pltpu.VMEM((NR, 128), jnp.float32),
            pltpu.VMEM((NR,), jnp.int32),
            pltpu.VMEM_SHARED((NR, 128), jnp.float32),
        ],
    )


NBUF = 2
NIS = 4


@functools.cache
def _make_edge_kernel(N_pad, C0, C1, H):
    R = N_pad // NS

    def body(ys_hbm, src_hbm, dst_hbm, zeros_hbm, out_hbm,
             sidx, didx, rows0, rows1, acc, gsem, ssem, isem):
        cid = lax.axis_index("c")
        sid = lax.axis_index("s")
        rowsb = (rows0, rows1)

        def make_pipeline(Cc, base):
            def start_idx(j):
                sl = j % NIS
                pltpu.async_copy(src_hbm.at[pl.ds(base + j * K, K)],
                                 sidx.at[sl], isem.at[sl])
                pltpu.async_copy(dst_hbm.at[pl.ds(base + j * K, K)],
                                 didx.at[sl], isem.at[sl])

            def wait_idx(j):
                sl = j % NIS
                pltpu.make_async_copy(src_hbm.at[pl.ds(0, K)], sidx.at[sl],
                                      isem.at[sl]).wait()
                pltpu.make_async_copy(dst_hbm.at[pl.ds(0, K)], didx.at[sl],
                                      isem.at[sl]).wait()

            def start_gather(b, j):
                pltpu.async_copy(ys_hbm.at[sidx.at[j % NIS]], rowsb[b],
                                 gsem.at[b])

            def wait_gather(b):
                pltpu.make_async_copy(ys_hbm.at[sidx.at[0]], rowsb[b],
                                      gsem.at[b]).wait()

            def start_scatter(b, j):
                pltpu.async_copy(rowsb[b], acc.at[didx.at[j % NIS]],
                                 ssem.at[b], add=True)

            def wait_scatter(b):
                pltpu.make_async_copy(rowsb[b], acc.at[didx.at[0]],
                                      ssem.at[b]).wait()

            for j in range(NIS):
                start_idx(j)
            for b in range(NBUF):
                wait_idx(b)
                start_gather(b, b)

            @pl.loop(0, Cc // NBUF)
            def _(q):
                for b in range(NBUF):
                    wait_gather(b)
                    start_scatter(b, q * NBUF + b)
                for b in range(NBUF):
                    jn = q * NBUF + NBUF + b
                    jf = q * NBUF + 2 * NBUF + b

                    @pl.when(jn < Cc)
                    def _():
                        wait_scatter(b)

                        @pl.when(jf < Cc)
                        def _():
                            start_idx(jf)

                        wait_idx(jn)
                        start_gather(b, jn)

            for b in range(NBUF):
                wait_scatter(b)

        pltpu.sync_copy(zeros_hbm, rows0)

        @pl.loop(0, R // K)
        def _(j):
            pltpu.sync_copy(rows0, acc.at[pl.ds(sid * R + j * K, K)])

        plsc.subcore_barrier()

        @pl.when(cid == 0)
        def _():
            make_pipeline(C0, sid * C0 * K)

        if C1 > 0:
            @pl.when(cid == 1)
            def _():
                make_pipeline(C1, NS * C0 * K + sid * C1 * K)

        plsc.subcore_barrier()

        @pl.loop(0, R // K)
        def _(j):
            r0 = sid * R + j * K
            pltpu.sync_copy(acc.at[pl.ds(r0, K)], rows0)
            pltpu.sync_copy(rows0, out_hbm.at[pl.ds(cid * N_pad + r0, K)])

    return pl.kernel(
        body,
        out_type=jax.ShapeDtypeStruct((NC * N_pad, H), jnp.float32),
        mesh=_sc_mesh(),
        scratch_types=[
            pltpu.VMEM((NIS, K), jnp.int32),
            pltpu.VMEM((NIS, K), jnp.int32),
            pltpu.VMEM((K, H), jnp.float32),
            pltpu.VMEM((K, H), jnp.float32),
            pltpu.VMEM_SHARED((N_pad, H), jnp.float32),
            pltpu.SemaphoreType.DMA((NBUF,)),
            pltpu.SemaphoreType.DMA((NBUF,)),
            pltpu.SemaphoreType.DMA((NIS,)),
        ],
    )


def _dinv_block(degp_ref, i, N):
    deg = (degp_ref[0] + degp_ref[1] + 1.0).reshape(BR, 1)
    row = i * BR + lax.broadcasted_iota(jnp.int32, (BR, 1), 0)
    return lax.rsqrt(deg) * (row < N).astype(jnp.float32)


@functools.cache
def _make_tc_first(N, N_pad, D, H):
    def body(x_ref, w_ref, degp_ref, o_ref):
        dinv = _dinv_block(degp_ref, pl.program_id(0), N)
        o_ref[...] = dinv * jnp.dot(x_ref[...], w_ref[...],
                                    preferred_element_type=jnp.float32)

    return pl.pallas_call(
        body,
        grid=(N_pad // BR,),
        in_specs=[
            pl.BlockSpec((BR, D), lambda i: (i, 0)),
            pl.BlockSpec((D, H), lambda i: (0, 0)),
            pl.BlockSpec((2, BR), lambda i: (0, i)),
        ],
        out_specs=pl.BlockSpec((BR, H), lambda i: (i, 0)),
        out_shape=jax.ShapeDtypeStruct((N_pad, H), jnp.float32),
    )


@functools.cache
def _make_tc_mid(N, N_pad, H):
    def body(p_ref, ys_ref, w_ref, b_ref, degp_ref, o_ref):
        dinv = _dinv_block(degp_ref, pl.program_id(0), N)
        agg = p_ref[0] + p_ref[1] + ys_ref[...]
        h = jnp.maximum(dinv * agg + b_ref[...], 0.0)
        o_ref[...] = dinv * jnp.dot(h, w_ref[...],
                                    preferred_element_type=jnp.float32)

    return pl.pallas_call(
        body,
        grid=(N_pad // BR,),
        in_specs=[
            pl.BlockSpec((2, BR, H), lambda i: (0, i, 0)),
            pl.BlockSpec((BR, H), lambda i: (i, 0)),
            pl.BlockSpec((H, H), lambda i: (0, 0)),
            pl.BlockSpec((1, H), lambda i: (0, 0)),
            pl.BlockSpec((2, BR), lambda i: (0, i)),
        ],
        out_specs=pl.BlockSpec((BR, H), lambda i: (i, 0)),
        out_shape=jax.ShapeDtypeStruct((N_pad, H), jnp.float32),
    )


@functools.cache
def _make_tc_final(N, N_pad, H):
    def body(p_ref, ys_ref, b_ref, degp_ref, o_ref):
        dinv = _dinv_block(degp_ref, pl.program_id(0), N)
        agg = p_ref[0] + p_ref[1] + ys_ref[...]
        o_ref[...] = dinv * agg + b_ref[...]

    return pl.pallas_call(
        body,
        grid=(N_pad // BR,),
        in_specs=[
            pl.BlockSpec((2, BR, H), lambda i: (0, i, 0)),
            pl.BlockSpec((BR, H), lambda i: (i, 0)),
            pl.BlockSpec((1, H), lambda i: (0, 0)),
            pl.BlockSpec((2, BR), lambda i: (0, i)),
        ],
        out_specs=pl.BlockSpec((BR, H), lambda i: (i, 0)),
        out_shape=jax.ShapeDtypeStruct((N_pad, H), jnp.float32),
    )


def kernel(x, edge_index, arg0, arg1, W0, b0, W1, b1, W2, b2):
    N, D = x.shape
    H = W0.shape[1]
    E = edge_index.shape[1]
    NW = NC * NS
    N_pad = _round_up(N + 1, NS * K)
    CT = _round_up(_round_up(E, NW * K) // (NW * K) * NC, 2 * NBUF)
    C0 = CT // 2
    C1 = CT - C0
    E_pad = CT * NS * K

    src = edge_index[0].astype(jnp.int32)
    dst = edge_index[1].astype(jnp.int32)
    if E_pad > E:
        padv = N + jnp.arange(E_pad - E, dtype=jnp.int32) % K
        src = jnp.concatenate([src, padv])
        dst = jnp.concatenate([dst, padv])

    x_p = jnp.pad(x.astype(jnp.float32), ((0, N_pad - N), (0, 0)))
    zeros_kh = jnp.zeros((K, H), jnp.float32)

    degp = _make_deg_kernel(N_pad, E_pad)(dst, zeros_kh).reshape(NC, N_pad)

    edge = _make_edge_kernel(N_pad, C0, C1, H)
    mid = _make_tc_mid(N, N_pad, H)

    ys = _make_tc_first(N, N_pad, D, H)(x_p, W0.astype(jnp.float32), degp)
    p = edge(ys, src, dst, zeros_kh).reshape(NC, N_pad, H)
    ys = mid(p, ys, W1.astype(jnp.float32), b0.reshape(1, H), degp)
    p = edge(ys, src, dst, zeros_kh).reshape(NC, N_pad, H)
    ys = mid(p, ys, W2.astype(jnp.float32), b1.reshape(1, H), degp)
    p = edge(ys, src, dst, zeros_kh).reshape(NC, N_pad, H)
    out = _make_tc_final(N, N_pad, H)(p, ys, b2.reshape(1, H), degp)

    return out[:N], arg1
